# Initial kernel scaffold; baseline (speedup 1.0000x reference)
import jax
import jax.numpy as jnp
from jax.experimental import pallas as pl


def kernel(x, edge_index, W1, b1, W2, b2, temp):
    raise NotImplementedError("write your pallas kernel here")


# speedup vs baseline: 16.1662x; 16.1662x over previous
import functools

import jax
import jax.numpy as jnp
import numpy as np
from jax import lax
from jax.experimental import pallas as pl
from jax.experimental.pallas import tpu as pltpu
from jax.experimental.pallas import tpu_sc as plsc

N = 10000
E = 320000
D = 128
HID = 64
C = 40
K = 10

NT = 16
NP = 10240
ROWS = NP // NT
CH = 128
NCH = ROWS // CH
SW = 128
SWT = 160
GRP = 8
NGRP = SWT // GRP
EP = NT * SWT * SW
L = 16
_OFF = (0, 16, 24)

_MLP_BLK = 1000


def _mlp_body(x_ref, w1_ref, b1_ref, w2_ref, b2_ref, o_ref):
    h = jnp.dot(x_ref[...], w1_ref[...], preferred_element_type=jnp.float32)
    h = jnp.maximum(h + b1_ref[...], 0.0)
    o_ref[...] = jnp.dot(h, w2_ref[...], preferred_element_type=jnp.float32) + b2_ref[...]


def _mlp(x, W1, b1, W2, b2):
    g = N // _MLP_BLK
    return pl.pallas_call(
        _mlp_body,
        grid=(g,),
        in_specs=[
            pl.BlockSpec((_MLP_BLK, D), lambda i: (i, 0)),
            pl.BlockSpec((D, HID), lambda i: (0, 0)),
            pl.BlockSpec((1, HID), lambda i: (0, 0)),
            pl.BlockSpec((HID, C), lambda i: (0, 0)),
            pl.BlockSpec((1, C), lambda i: (0, 0)),
        ],
        out_specs=pl.BlockSpec((_MLP_BLK, C), lambda i: (i, 0)),
        out_shape=jax.ShapeDtypeStruct((N, C), jnp.float32),
    )(x, W1, b1, W2, b2)


def _lsm_body(h_ref, o_ref):
    v = h_ref[...]
    m = jnp.max(v, axis=1, keepdims=True)
    e = jnp.exp(v - m)
    s = jnp.sum(e, axis=1, keepdims=True)
    o_ref[...] = v - m - jnp.log(s)


def _lsm(hidden_pad):
    g = N // _MLP_BLK
    return pl.pallas_call(
        _lsm_body,
        grid=(g,),
        in_specs=[pl.BlockSpec((_MLP_BLK, C), lambda i: (i, 0))],
        out_specs=pl.BlockSpec((_MLP_BLK, C), lambda i: (i, 0)),
        out_shape=jax.ShapeDtypeStruct((N, C), jnp.float32),
    )(hidden_pad)


def _rsqrt16(v):
    i = lax.bitcast_convert_type(v, jnp.int32)
    i = jnp.int32(0x5F3759DF) - lax.shift_right_arithmetic(i, 1)
    r = lax.bitcast_convert_type(i, jnp.float32)
    for _ in range(3):
        r = r * (1.5 - 0.5 * v * r * r)
    return r


def _prop_body(h_hbm, row_hbm, col_hbm, temp_hbm, out_hbm,
               y_sh, s_sh, deg_sh,
               rowi, coli, msg, one_v, abuf, bbuf, zbuf,
               hidl, d2s, degl, tmp_v):
    tid = lax.axis_index("s")
    rbase = tid * ROWS
    wbase = tid * SWT

    ones16 = jnp.ones((L,), jnp.float32)
    zeros16 = jnp.zeros((L,), jnp.float32)

    pltpu.sync_copy(temp_hbm, tmp_v)

    def _init_one(i, c):
        one_v[pl.ds(i * L, L)] = ones16
        return c
    lax.fori_loop(0, SW // L, _init_one, 0)

    def _init_deg(i, c):
        degl[pl.ds(i * L, L)] = ones16
        return c
    lax.fori_loop(0, ROWS // L, _init_deg, 0)
    pltpu.sync_copy(degl, deg_sh.at[pl.ds(rbase, ROWS)])

    def _init_z(r, c):
        for o in _OFF:
            zbuf[r, pl.ds(o, L)] = zeros16
        return c
    lax.fori_loop(0, CH, _init_z, 0)
    for c in range(NCH):
        pltpu.sync_copy(zbuf, s_sh.at[pl.ds(rbase + c * CH, CH)])

    plsc.subcore_barrier()

    def _deg_grp(g, c):
        pltpu.sync_copy(col_hbm.at[pl.ds(wbase + g * GRP, GRP)], coli)
        for b in range(GRP):
            pltpu.sync_copy(one_v, deg_sh.at[coli.at[b]], add=True)
        return c
    lax.fori_loop(0, NGRP, _deg_grp, 0)

    plsc.subcore_barrier()

    pltpu.sync_copy(deg_sh.at[pl.ds(rbase, ROWS)], degl)

    def _build_d2(i, c):
        dvec = degl[pl.ds(i * L, L)]
        d2s[pl.ds(i * L, L)] = 1.0 / dvec
        return c
    lax.fori_loop(0, ROWS // L, _build_d2, 0)

    tvec = tmp_v[pl.ds(0, L)]
    t0 = tvec[0]

    def _y0_chunk(c, cc):
        rows = pl.ds(rbase + c * CH, CH)
        pltpu.sync_copy(h_hbm.at[rows], abuf)

        def _y0_blk(i, dd):
            d16 = d2s[pl.ds(c * CH + i * L, L)]
            sq16 = d16 * _rsqrt16(d16)
            for m in range(L):
                r = i * L + m
                sq = sq16[m]
                vs = [abuf[r, pl.ds(o, L)] * sq for o in _OFF]
                for o, v in zip(_OFF, vs):
                    abuf[r, pl.ds(o, L)] = v
                    hidl[c * CH + r, pl.ds(o, L)] = t0 * v
            return dd
        lax.fori_loop(0, CH // L, _y0_blk, 0)
        pltpu.sync_copy(abuf, y_sh.at[rows])
        return cc
    lax.fori_loop(0, NCH, _y0_chunk, 0)

    plsc.subcore_barrier()

    def _round(k, carry):
        def _edge_grp(g, c):
            gw = wbase + g * GRP
            pltpu.sync_copy(row_hbm.at[pl.ds(gw, GRP)], rowi)
            pltpu.sync_copy(col_hbm.at[pl.ds(gw, GRP)], coli)
            for b in range(GRP):
                pltpu.sync_copy(y_sh.at[rowi.at[b]], msg)
                pltpu.sync_copy(msg, s_sh.at[coli.at[b]], add=True)
            return c
        lax.fori_loop(0, NGRP, _edge_grp, 0)

        plsc.subcore_barrier()

        tk = jnp.float32(0.0)
        for m in range(1, K + 1):
            tk = jnp.where(k + 1 == m, tvec[m], tk)

        def _comb_chunk(c, cc):
            rows = pl.ds(rbase + c * CH, CH)
            pltpu.sync_copy(y_sh.at[rows], abuf)
            pltpu.sync_copy(s_sh.at[rows], bbuf)
            pltpu.sync_copy(zbuf, s_sh.at[rows])

            def _comb_blk(i, dd):
                d16 = d2s[pl.ds(c * CH + i * L, L)]
                for m in range(L):
                    r = i * L + m
                    d2 = d16[m]
                    yns = [(abuf[r, pl.ds(o, L)] + bbuf[r, pl.ds(o, L)]) * d2
                           for o in _OFF]
                    hvs = [hidl[c * CH + r, pl.ds(o, L)] for o in _OFF]
                    for o, yn, hv in zip(_OFF, yns, hvs):
                        abuf[r, pl.ds(o, L)] = yn
                        hidl[c * CH + r, pl.ds(o, L)] = hv + tk * yn
                return dd
            lax.fori_loop(0, CH // L, _comb_blk, 0)
            pltpu.sync_copy(abuf, y_sh.at[rows])
            return cc
        lax.fori_loop(0, NCH, _comb_chunk, 0)

        plsc.subcore_barrier()
        return carry
    lax.fori_loop(0, K, _round, 0)

    def _out_chunk(c, cc):
        def _out_blk(i, dd):
            d16 = d2s[pl.ds(c * CH + i * L, L)]
            rs16 = _rsqrt16(d16)
            for m in range(L):
                r = i * L + m
                rs = rs16[m]
                vs = [hidl[c * CH + r, pl.ds(o, L)] * rs for o in _OFF]
                for o, v in zip(_OFF, vs):
                    abuf[r, pl.ds(o, L)] = v
            return dd
        lax.fori_loop(0, CH // L, _out_blk, 0)
        pltpu.sync_copy(abuf, out_hbm.at[pl.ds(rbase + c * CH, CH)])
        return cc
    lax.fori_loop(0, NCH, _out_chunk, 0)


_prop = pl.kernel(
    _prop_body,
    out_type=jax.ShapeDtypeStruct((NP, C), jnp.float32),
    mesh=plsc.VectorSubcoreMesh(core_axis_name="c", subcore_axis_name="s",
                                num_cores=1, num_subcores=NT),
    compiler_params=pltpu.CompilerParams(use_tc_tiling_on_sc=False),
    scratch_types=[
        pltpu.VMEM_SHARED((NP, C), jnp.float32),
        pltpu.VMEM_SHARED((NP, C), jnp.float32),
        pltpu.VMEM_SHARED((NP,), jnp.float32),
        pltpu.VMEM((GRP, SW), jnp.int32),
        pltpu.VMEM((GRP, SW), jnp.int32),
        pltpu.VMEM((SW, C), jnp.float32),
        pltpu.VMEM((SW,), jnp.float32),
        pltpu.VMEM((CH, C), jnp.float32),
        pltpu.VMEM((CH, C), jnp.float32),
        pltpu.VMEM((CH, C), jnp.float32),
        pltpu.VMEM((ROWS, C), jnp.float32),
        pltpu.VMEM((ROWS,), jnp.float32),
        pltpu.VMEM((ROWS,), jnp.float32),
        pltpu.VMEM((L,), jnp.float32),
    ],
)


def kernel(x, edge_index, W1, b1, W2, b2, temp):
    h = _mlp(x, W1, b1.reshape(1, HID), W2, b2.reshape(1, C))
    h_pad = jnp.pad(h, ((0, NP - N), (0, 0)))
    pad = jnp.full((EP - E,), N, jnp.int32)
    row2d = jnp.concatenate([edge_index[0], pad]).reshape(EP // SW, SW)
    col2d = jnp.concatenate([edge_index[1], pad]).reshape(EP // SW, SW)
    temp16 = jnp.pad(temp, (0, L - (K + 1)))
    hidden = _prop(h_pad, row2d, col2d, temp16)
    return _lsm(hidden)
